# baseline (speedup 1.0000x reference)
import jax
import jax.numpy as jnp
from jax import lax
from jax.experimental import pallas as pl
from jax.experimental.pallas import tpu as pltpu
from jax.experimental.pallas import tpu_sc as plsc

B = 4
S = 2048
C = 2048
POOL = 64
LEN = 5
TOPK = 8
HEAD = TOPK * LEN
CHUNK = 512
JX = S // CHUNK


def _route(sums, pk, sim_ref, idx_ref, idx40_ref, rsum_ref):
    xm = sums * (1.0 / S)
    xn = xm * lax.rsqrt(jnp.maximum(jnp.sum(xm * xm, axis=1, keepdims=True), 1e-12))
    pkn = pk * lax.rsqrt(jnp.maximum(jnp.sum(pk * pk, axis=1, keepdims=True), 1e-12))
    sim = lax.dot_general(
        xn, pkn, (((1,), (1,)), ((), ())),
        preferred_element_type=jnp.float32,
        precision=lax.Precision.DEFAULT,
    )
    sim_ref[...] = sim

    col = lax.broadcasted_iota(jnp.int32, (B, POOL), 1)
    sub = lax.broadcasted_iota(jnp.int32, (B, LEN), 1)
    masked = sim
    acc = jnp.float32(0.0)
    for t in range(TOPK):
        m = jnp.max(masked, axis=1, keepdims=True)
        acc = acc + jnp.sum(m)
        it = jnp.min(jnp.where(masked == m, col, POOL), axis=1)
        idx_ref[:, t : t + 1] = it[:, None]
        idx40_ref[:, LEN * t : LEN * (t + 1)] = it[:, None] * LEN + sub
        masked = jnp.where(col == it[:, None], -jnp.inf, masked)
    rsum_ref[...] = jnp.full((1, 1), acc * (1.0 / B), jnp.float32)


def _bulk_body(x_ref, pk_ref, out_ref, sim_ref, idx_ref, idx40_ref, rsum_ref,
               sums_ref, carry_ref):
    b = pl.program_id(0)
    j = pl.program_id(1)

    @pl.when(j < JX)
    def _():
        part = jnp.sum(x_ref[0], axis=0)[None, None, :]

        for i in range(B):
            @pl.when((b == i) & (j == 0))
            def _(i=i):
                sums_ref[i : i + 1] = part

            @pl.when((b == i) & (j > 0))
            def _(i=i):
                sums_ref[i : i + 1] = sums_ref[i : i + 1] + part

        out_ref[0, :HEAD, :] = carry_ref[...]
        out_ref[0, HEAD:, :] = x_ref[0, : CHUNK - HEAD, :]
        carry_ref[...] = x_ref[0, CHUNK - HEAD :, :]

    @pl.when(j == JX)
    def _():
        out_ref[0, :HEAD, :] = carry_ref[...]

    @pl.when((b == B - 1) & (j == JX))
    def _():
        _route(sums_ref[:, 0, :], pk_ref[...], sim_ref, idx_ref, idx40_ref,
               rsum_ref)


def _head_body(idx40_hbm, prompt_hbm, out_hbm, idx_v, rows_v, sem):
    wid = lax.axis_index("s") * 2 + lax.axis_index("c")

    @pl.when(wid < B)
    def _():
        b = wid
        pltpu.sync_copy(idx40_hbm.at[b], idx_v)
        pltpu.async_copy(prompt_hbm.at[idx_v], rows_v, sem).wait()
        pltpu.sync_copy(rows_v, out_hbm.at[b, pl.ds(0, HEAD), :])


def kernel(x_embed, prompt, prompt_key):
    big, sim, idx, idx40, rsum = pl.pallas_call(
        _bulk_body,
        grid=(B, JX + 1),
        in_specs=[
            pl.BlockSpec((1, CHUNK, C), lambda b, j: (b, jnp.minimum(j, JX - 1), 0)),
            pl.BlockSpec((POOL, C), lambda b, j: (0, 0)),
        ],
        out_specs=[
            pl.BlockSpec((1, CHUNK, C), lambda b, j: (b, j, 0)),
            pl.BlockSpec((B, POOL), lambda b, j: (0, 0)),
            pl.BlockSpec((B, TOPK), lambda b, j: (0, 0)),
            pl.BlockSpec((B, HEAD), lambda b, j: (0, 0)),
            pl.BlockSpec((1, 1), lambda b, j: (0, 0)),
        ],
        out_shape=[
            jax.ShapeDtypeStruct((B, HEAD + S, C), jnp.float32),
            jax.ShapeDtypeStruct((B, POOL), jnp.float32),
            jax.ShapeDtypeStruct((B, TOPK), jnp.int32),
            jax.ShapeDtypeStruct((B, HEAD), jnp.int32),
            jax.ShapeDtypeStruct((1, 1), jnp.float32),
        ],
        scratch_shapes=[
            pltpu.VMEM((B, 1, C), jnp.float32),
            pltpu.VMEM((HEAD, C), jnp.float32),
        ],
    )(x_embed, prompt_key)

    mesh = plsc.VectorSubcoreMesh(core_axis_name="c", subcore_axis_name="s")
    gather_head = pl.kernel(
        _head_body,
        out_type=(),
        mesh=mesh,
        scratch_types=[
            pltpu.VMEM((HEAD,), jnp.int32),
            pltpu.VMEM((HEAD, C), jnp.float32),
            pltpu.SemaphoreType.DMA,
        ],
    )
    big_ref = jax.new_ref(big)
    gather_head(idx40, prompt.reshape(POOL * LEN, C), big_ref)
    prompted = jax.freeze(big_ref)

    return (prompted, rsum[0, 0], sim, idx)


# speedup vs baseline: 1.1641x; 1.1641x over previous
import jax
import jax.numpy as jnp
from jax import lax
from jax.experimental import pallas as pl
from jax.experimental.pallas import tpu as pltpu

B = 4
S = 2048
C = 2048
POOL = 64
LEN = 5
TOPK = 8
HEAD = TOPK * LEN
BLK = 232
NB = 9
REST = BLK - HEAD
NSTEP = B * NB + 1


def _routing(sums_row, pk, bb, sim_ref, idx_ref, rsum_ref, head_ref, rs_ref):
    xm = sums_row * (1.0 / S)
    xn = xm * lax.rsqrt(jnp.maximum(jnp.sum(xm * xm, axis=1, keepdims=True), 1e-12))
    pkn = pk * lax.rsqrt(jnp.maximum(jnp.sum(pk * pk, axis=1, keepdims=True), 1e-12))
    sim = lax.dot_general(
        xn, pkn, (((1,), (1,)), ((), ())),
        preferred_element_type=jnp.float32,
        precision=lax.Precision.DEFAULT,
    )

    col = lax.broadcasted_iota(jnp.int32, (1, POOL), 1)
    col8 = lax.broadcasted_iota(jnp.int32, (1, TOPK), 1)
    r40 = lax.broadcasted_iota(jnp.int32, (HEAD, 1), 0)
    masked = sim
    acc = jnp.float32(0.0)
    idxrow = jnp.zeros((1, TOPK), jnp.int32)
    it40 = jnp.zeros((HEAD, 1), jnp.int32)
    for t in range(TOPK):
        m = jnp.max(masked, axis=1, keepdims=True)
        acc = acc + m[0, 0]
        it = jnp.min(jnp.where(masked == m, col, POOL), axis=1)
        it11 = it[:, None]
        idxrow = jnp.where(col8 == t, it11, idxrow)
        it40 = jnp.where(r40 // LEN == t, it11, it40)
        masked = jnp.where(col == it11, -jnp.inf, masked)

    c320 = lax.broadcasted_iota(jnp.int32, (HEAD, POOL * LEN), 1)
    oh = ((c320 // LEN == it40) & (c320 % LEN == r40 % LEN)).astype(jnp.float32)
    head_ref[...] = oh

    for i in range(B):
        @pl.when(bb == i)
        def _(i=i):
            sim_ref[i : i + 1, :] = sim
            idx_ref[i : i + 1, :] = idxrow

    @pl.when(bb == 0)
    def _():
        rs_ref[...] = jnp.full((1, 1), acc, jnp.float32)

    @pl.when(bb > 0)
    def _():
        rs_ref[...] = rs_ref[...] + acc

    @pl.when(bb == B - 1)
    def _():
        rsum_ref[...] = rs_ref[...] * (1.0 / B)


def _body(x_ref, pk_ref, p2_ref, out_ref, sim_ref, idx_ref, rsum_ref,
          sums_ref, carry_ref, stash_ref, oh_ref, rs_ref):
    s = pl.program_id(0)
    bb = s // NB
    m = lax.rem(s, NB)

    @pl.when(m != 0)
    def _():
        out_ref[0, :HEAD, :] = carry_ref[...]
        out_ref[0, HEAD:, :] = x_ref[0, :REST, :]

    @pl.when((m == 0) & (s > 0))
    def _():
        out_ref[0, :HEAD, :] = lax.dot_general(
            oh_ref[...], p2_ref[...], (((1,), (0,)), ((), ())),
            preferred_element_type=jnp.float32,
            precision=lax.Precision.HIGHEST,
        )
        out_ref[0, HEAD:, :] = stash_ref[...]

    @pl.when((m == 0) & (s < NSTEP - 1))
    def _():
        stash_ref[...] = x_ref[0, :REST, :]

    carry_ref[...] = x_ref[0, REST:, :]

    @pl.when(s < NSTEP - 1)
    def _():
        part_lo = jnp.sum(x_ref[0, :REST, :], axis=0)
        part_hi = jnp.sum(x_ref[0, REST:, :], axis=0)
        part = part_lo + jnp.where(m == NB - 1, 0.0, part_hi)

        @pl.when(m == 0)
        def _():
            sums_ref[...] = part[None, :]

        @pl.when(m != 0)
        def _():
            sums_ref[...] = sums_ref[...] + part[None, :]

        @pl.when(m == NB - 1)
        def _():
            _routing(sums_ref[...], pk_ref[...], bb, sim_ref, idx_ref,
                     rsum_ref, oh_ref, rs_ref)


def kernel(x_embed, prompt, prompt_key):
    def x_map(s):
        t = jnp.minimum(s, NSTEP - 2)
        return (t // NB, lax.rem(t, NB), 0)

    def out_map(s):
        bo = s // NB
        mo = lax.rem(s, NB)
        b_o = jnp.where(mo == 0, bo - 1, bo)
        return (jnp.maximum(b_o, 0), jnp.where(s == 0, 1, mo), 0)

    big, sim, idx, rsum = pl.pallas_call(
        _body,
        grid=(NSTEP,),
        in_specs=[
            pl.BlockSpec((1, BLK, C), x_map),
            pl.BlockSpec((POOL, C), lambda s: (0, 0)),
            pl.BlockSpec((POOL * LEN, C), lambda s: (0, 0)),
        ],
        out_specs=[
            pl.BlockSpec((1, BLK, C), out_map),
            pl.BlockSpec((B, POOL), lambda s: (0, 0)),
            pl.BlockSpec((B, TOPK), lambda s: (0, 0)),
            pl.BlockSpec((1, 1), lambda s: (0, 0)),
        ],
        out_shape=[
            jax.ShapeDtypeStruct((B, HEAD + S, C), jnp.float32),
            jax.ShapeDtypeStruct((B, POOL), jnp.float32),
            jax.ShapeDtypeStruct((B, TOPK), jnp.int32),
            jax.ShapeDtypeStruct((1, 1), jnp.float32),
        ],
        scratch_shapes=[
            pltpu.VMEM((1, C), jnp.float32),
            pltpu.VMEM((HEAD, C), jnp.float32),
            pltpu.VMEM((REST, C), jnp.float32),
            pltpu.VMEM((HEAD, POOL * LEN), jnp.float32),
            pltpu.VMEM((1, 1), jnp.float32),
        ],
    )(x_embed, prompt_key, prompt.reshape(POOL * LEN, C))

    return (big, rsum[0, 0], sim, idx)


# speedup vs baseline: 1.4732x; 1.2655x over previous
import jax
import jax.numpy as jnp
from jax import lax
from jax.experimental import pallas as pl
from jax.experimental.pallas import tpu as pltpu

B = 4
S = 2048
C = 2048
POOL = 64
LEN = 5
TOPK = 8
HEAD = TOPK * LEN
BLK = 696
NB = 3
REST = BLK - HEAD
NSTEP = B * NB + 1


def _routing(sums_row, pk, bb, sim_ref, idx_ref, rsum_ref, head_ref, rs_ref):
    xm = sums_row * (1.0 / S)
    xn = xm * lax.rsqrt(jnp.maximum(jnp.sum(xm * xm, axis=1, keepdims=True), 1e-12))
    pkn = pk * lax.rsqrt(jnp.maximum(jnp.sum(pk * pk, axis=1, keepdims=True), 1e-12))
    sim = lax.dot_general(
        xn, pkn, (((1,), (1,)), ((), ())),
        preferred_element_type=jnp.float32,
        precision=lax.Precision.DEFAULT,
    )

    col = lax.broadcasted_iota(jnp.int32, (1, POOL), 1)
    col8 = lax.broadcasted_iota(jnp.int32, (1, TOPK), 1)
    r40 = lax.broadcasted_iota(jnp.int32, (HEAD, 1), 0)
    masked = sim
    acc = jnp.float32(0.0)
    idxrow = jnp.zeros((1, TOPK), jnp.int32)
    it40 = jnp.zeros((HEAD, 1), jnp.int32)
    for t in range(TOPK):
        m = jnp.max(masked, axis=1, keepdims=True)
        acc = acc + m[0, 0]
        it = jnp.min(jnp.where(masked == m, col, POOL), axis=1)
        it11 = it[:, None]
        idxrow = jnp.where(col8 == t, it11, idxrow)
        it40 = jnp.where(r40 // LEN == t, it11, it40)
        masked = jnp.where(col == it11, -jnp.inf, masked)

    c320 = lax.broadcasted_iota(jnp.int32, (HEAD, POOL * LEN), 1)
    oh = ((c320 // LEN == it40) & (c320 % LEN == r40 % LEN)).astype(jnp.float32)
    head_ref[...] = oh

    for i in range(B):
        @pl.when(bb == i)
        def _(i=i):
            sim_ref[i : i + 1, :] = sim
            idx_ref[i : i + 1, :] = idxrow

    @pl.when(bb == 0)
    def _():
        rs_ref[...] = jnp.full((1, 1), acc, jnp.float32)

    @pl.when(bb > 0)
    def _():
        rs_ref[...] = rs_ref[...] + acc

    @pl.when(bb == B - 1)
    def _():
        rsum_ref[...] = rs_ref[...] * (1.0 / B)


def _body(x_ref, pk_ref, p2_ref, out_ref, sim_ref, idx_ref, rsum_ref,
          sums_ref, carry_ref, stash_ref, oh_ref, rs_ref):
    s = pl.program_id(0)
    bb = s // NB
    m = lax.rem(s, NB)

    @pl.when(m != 0)
    def _():
        out_ref[0, :HEAD, :] = carry_ref[...]
        out_ref[0, HEAD:, :] = x_ref[0, :REST, :]

    @pl.when((m == 0) & (s > 0))
    def _():
        out_ref[0, :HEAD, :] = lax.dot_general(
            oh_ref[...], p2_ref[...], (((1,), (0,)), ((), ())),
            preferred_element_type=jnp.float32,
            precision=lax.Precision.HIGHEST,
        )
        out_ref[0, HEAD:, :] = stash_ref[...]

    @pl.when((m == 0) & (s < NSTEP - 1))
    def _():
        stash_ref[...] = x_ref[0, :REST, :]

    carry_ref[...] = x_ref[0, REST:, :]

    @pl.when(s < NSTEP - 1)
    def _():
        part_lo = jnp.sum(x_ref[0, :REST, :], axis=0)
        part_hi = jnp.sum(x_ref[0, REST:, :], axis=0)
        part = part_lo + jnp.where(m == NB - 1, 0.0, part_hi)

        @pl.when(m == 0)
        def _():
            sums_ref[...] = part[None, :]

        @pl.when(m != 0)
        def _():
            sums_ref[...] = sums_ref[...] + part[None, :]

        @pl.when(m == NB - 1)
        def _():
            _routing(sums_ref[...], pk_ref[...], bb, sim_ref, idx_ref,
                     rsum_ref, oh_ref, rs_ref)


def kernel(x_embed, prompt, prompt_key):
    def x_map(s):
        t = jnp.minimum(s, NSTEP - 2)
        return (t // NB, lax.rem(t, NB), 0)

    def out_map(s):
        bo = s // NB
        mo = lax.rem(s, NB)
        b_o = jnp.where(mo == 0, bo - 1, bo)
        return (jnp.maximum(b_o, 0), jnp.where(s == 0, 1, mo), 0)

    big, sim, idx, rsum = pl.pallas_call(
        _body,
        grid=(NSTEP,),
        in_specs=[
            pl.BlockSpec((1, BLK, C), x_map),
            pl.BlockSpec((POOL, C), lambda s: (0, 0)),
            pl.BlockSpec((POOL * LEN, C), lambda s: (0, 0)),
        ],
        out_specs=[
            pl.BlockSpec((1, BLK, C), out_map),
            pl.BlockSpec((B, POOL), lambda s: (0, 0)),
            pl.BlockSpec((B, TOPK), lambda s: (0, 0)),
            pl.BlockSpec((1, 1), lambda s: (0, 0)),
        ],
        out_shape=[
            jax.ShapeDtypeStruct((B, HEAD + S, C), jnp.float32),
            jax.ShapeDtypeStruct((B, POOL), jnp.float32),
            jax.ShapeDtypeStruct((B, TOPK), jnp.int32),
            jax.ShapeDtypeStruct((1, 1), jnp.float32),
        ],
        scratch_shapes=[
            pltpu.VMEM((1, C), jnp.float32),
            pltpu.VMEM((HEAD, C), jnp.float32),
            pltpu.VMEM((REST, C), jnp.float32),
            pltpu.VMEM((HEAD, POOL * LEN), jnp.float32),
            pltpu.VMEM((1, 1), jnp.float32),
        ],
    )(x_embed, prompt_key, prompt.reshape(POOL * LEN, C))

    return (big, rsum[0, 0], sim, idx)
